# baseline (speedup 1.0000x reference)
import functools

import jax
import jax.numpy as jnp
from jax import lax
from jax.experimental import pallas as pl
from jax.experimental.pallas import tpu as pltpu
from jax.experimental.pallas import tpu_sc as plsc

_S = 64.0
_M = 0.5


def _sc_fix_vals(cosine, label, cms, sms):
    B, C = cosine.shape
    info = plsc.get_sparse_core_info()
    NC, NS, L = info.num_cores, info.num_subcores, info.num_lanes
    NW = NC * NS
    rpw = B // NW
    mesh = plsc.VectorSubcoreMesh(core_axis_name="c", subcore_axis_name="s")

    @functools.partial(
        pl.kernel,
        mesh=mesh,
        compiler_params=pltpu.CompilerParams(needs_layout_passes=False),
        out_type=jax.ShapeDtypeStruct((B,), jnp.float32),
        scratch_types=[
            pltpu.VMEM((rpw,), jnp.int32),
            pltpu.VMEM((rpw, 8, 128), jnp.float32),
            pltpu.VMEM((rpw,), jnp.float32),
            pltpu.VMEM((rpw,), jnp.float32),
            pltpu.VMEM((rpw,), jnp.float32),
            pltpu.SemaphoreType.DMA,
        ],
    )
    def k(cos_hbm, lab_hbm, cms_hbm, sms_hbm, out_hbm,
          lab_v, win_v, cm_v, sm_v, fix_v, sem):
        wid = lax.axis_index("s") * NC + lax.axis_index("c")
        base = wid * rpw
        pltpu.sync_copy(lab_hbm.at[pl.ds(base, rpw)], lab_v)
        pltpu.sync_copy(cms_hbm.at[pl.ds(base, rpw)], cm_v)
        pltpu.sync_copy(sms_hbm.at[pl.ds(base, rpw)], sm_v)
        lanes = lax.iota(jnp.int32, L)
        cps = []
        for j in range(rpw):
            grp = lab_v[pl.ds((j // L) * L, L)]
            c = jnp.sum(jnp.where(lanes == (j % L), grp, 0))
            cst = pl.multiple_of((c >> 7) << 7, 128)
            rst = pl.multiple_of(((base + j) >> 3) << 3, 8)
            cp = pltpu.make_async_copy(
                cos_hbm.at[pl.ds(rst, 8), pl.ds(cst, 128)], win_v.at[j], sem)
            cp.start()
            cps.append(cp)
        for cp in cps:
            cp.wait()
        for g in range(rpw // L):
            sl = pl.ds(g * L, L)
            lab = lab_v[sl]
            coff = lab & 127
            rows = lax.iota(jnp.int32, L) + g * L
            roff = (rows + base) & 7
            x = plsc.load_gather(win_v, [rows, roff, coff])
            x = jnp.minimum(jnp.maximum(x, -1.0), 1.0)
            a = jnp.maximum(1.0 - x * x, 0.0)
            y = 0.27 + 0.77 * a
            for _ in range(4):
                y = 0.5 * (y + a / y)
            fix_v[sl] = x * cm_v[sl] - y * sm_v[sl]
        pltpu.sync_copy(fix_v, out_hbm.at[pl.ds(base, rpw)])

    return k(cosine, label, cms, sms)


def _tc_apply(cosine, label2d, fix2d, sB=8, K=2, P=4):
    B, C = cosine.shape
    rows_per_step = sB * K * P
    nsteps = B // rows_per_step

    def body(cos_hbm, lab_ref, fix_ref, out_hbm, *scratch):
        bufs_in = scratch[:P]
        bufs_out = scratch[P:2 * P]
        in_sems = scratch[2 * P:3 * P]
        out_sems = scratch[3 * P:4 * P]
        g = pl.program_id(0)

        def in_cp(row0, k, p):
            return pltpu.make_async_copy(
                cos_hbm.at[pl.ds(row0, sB)], bufs_in[p].at[k],
                in_sems[p].at[k])

        def out_cp(row0, k, p):
            return pltpu.make_async_copy(
                bufs_out[p].at[k], out_hbm.at[pl.ds(row0, sB)],
                out_sems[p].at[k])

        @pl.when(g == 0)
        def _prime():
            for k in range(K):
                for p in range(P):
                    in_cp((k * P + p) * sB, k, p).start()

        for k in range(K):
            for p in range(P):
                row0 = ((g * K + k) * P + p) * sB
                in_cp(row0, k, p).wait()

                @pl.when(g > 0)
                def _drain_out():
                    out_cp(row0 - rows_per_step, k, p).wait()

                r0 = (k * P + p) * sB
                x = bufs_in[p][k]
                lab = lab_ref[pl.ds(r0, sB), :]
                fv = fix_ref[pl.ds(r0, sB), :]
                cols = lax.broadcasted_iota(jnp.int32, (1, C), 1)
                bufs_out[p][k] = jnp.where(cols == lab, fv, x * _S)
                out_cp(row0, k, p).start()

                @pl.when(g + 1 < nsteps)
                def _prefetch():
                    in_cp(row0 + rows_per_step, k, p).start()

        @pl.when(g == nsteps - 1)
        def _drain_all():
            for k in range(K):
                for p in range(P):
                    out_cp(((g * K + k) * P + p) * sB, k, p).wait()

    return pl.pallas_call(
        body,
        grid=(nsteps,),
        in_specs=[
            pl.BlockSpec(memory_space=pl.ANY),
            pl.BlockSpec((rows_per_step, 1), lambda i: (i, 0)),
            pl.BlockSpec((rows_per_step, 1), lambda i: (i, 0)),
        ],
        out_specs=pl.BlockSpec(memory_space=pl.ANY),
        out_shape=jax.ShapeDtypeStruct((B, C), jnp.float32),
        scratch_shapes=(
            [pltpu.VMEM((K, sB, C), jnp.float32) for _ in range(2 * P)]
            + [pltpu.SemaphoreType.DMA((K,)) for _ in range(2 * P)]
        ),
    )(cosine, label2d, fix2d)


def kernel(cosine, label):
    B, C = cosine.shape
    margin = jax.random.normal(jax.random.key(42), (B,), jnp.float32) * 0.1 + _M
    cms = jnp.cos(margin) * _S
    sms = jnp.sin(margin) * _S
    fix = _sc_fix_vals(cosine, label, cms, sms)
    return _tc_apply(cosine, label.reshape(B, 1), fix.reshape(B, 1))


# speedup vs baseline: 1.0016x; 1.0016x over previous
import functools

import jax
import jax.numpy as jnp
from jax import lax
from jax.experimental import pallas as pl
from jax.experimental.pallas import tpu as pltpu
from jax.experimental.pallas import tpu_sc as plsc

_S = 64.0
_M = 0.5


def _sc_fix_vals(cosine, label, cms, sms):
    B, C = cosine.shape
    info = plsc.get_sparse_core_info()
    NC, NS, L = info.num_cores, info.num_subcores, info.num_lanes
    NW = NC * NS
    rpw = B // NW
    mesh = plsc.VectorSubcoreMesh(core_axis_name="c", subcore_axis_name="s")

    @functools.partial(
        pl.kernel,
        mesh=mesh,
        compiler_params=pltpu.CompilerParams(needs_layout_passes=False),
        out_type=jax.ShapeDtypeStruct((B,), jnp.float32),
        scratch_types=[
            pltpu.VMEM((rpw,), jnp.int32),
            pltpu.VMEM((rpw, 8, 128), jnp.float32),
            pltpu.VMEM((rpw,), jnp.float32),
            pltpu.VMEM((rpw,), jnp.float32),
            pltpu.VMEM((rpw,), jnp.float32),
            pltpu.SemaphoreType.DMA,
        ],
    )
    def k(cos_hbm, lab_hbm, cms_hbm, sms_hbm, out_hbm,
          lab_v, win_v, cm_v, sm_v, fix_v, sem):
        wid = lax.axis_index("s") * NC + lax.axis_index("c")
        base = wid * rpw
        pltpu.sync_copy(lab_hbm.at[pl.ds(base, rpw)], lab_v)
        pltpu.sync_copy(cms_hbm.at[pl.ds(base, rpw)], cm_v)
        pltpu.sync_copy(sms_hbm.at[pl.ds(base, rpw)], sm_v)
        lanes = lax.iota(jnp.int32, L)
        cps = []
        for j in range(rpw):
            grp = lab_v[pl.ds((j // L) * L, L)]
            c = jnp.sum(jnp.where(lanes == (j % L), grp, 0))
            cst = pl.multiple_of((c >> 7) << 7, 128)
            rst = pl.multiple_of(((base + j) >> 3) << 3, 8)
            cp = pltpu.make_async_copy(
                cos_hbm.at[pl.ds(rst, 8), pl.ds(cst, 128)], win_v.at[j], sem)
            cp.start()
            cps.append(cp)
        for cp in cps:
            cp.wait()
        for g in range(rpw // L):
            sl = pl.ds(g * L, L)
            lab = lab_v[sl]
            coff = lab & 127
            rows = lax.iota(jnp.int32, L) + g * L
            roff = (rows + base) & 7
            x = plsc.load_gather(win_v, [rows, roff, coff])
            x = jnp.minimum(jnp.maximum(x, -1.0), 1.0)
            a = jnp.maximum(1.0 - x * x, 0.0)
            y = 0.27 + 0.77 * a
            for _ in range(4):
                y = 0.5 * (y + a / y)
            fix_v[sl] = x * cm_v[sl] - y * sm_v[sl]
        pltpu.sync_copy(fix_v, out_hbm.at[pl.ds(base, rpw)])

    return k(cosine, label, cms, sms)


def _tc_apply(cosine, label2d, fix2d, sB=8, K=2, P=4):
    B, C = cosine.shape
    rows_per_step = sB * K * P
    nsteps = B // rows_per_step

    def body(cos_hbm, lab_ref, fix_ref, out_hbm, *scratch):
        bufs_in = scratch[:P]
        bufs_out = scratch[P:2 * P]
        in_sems = scratch[2 * P:3 * P]
        out_sems = scratch[3 * P:4 * P]
        g = pl.program_id(0)

        def in_cp(row0, k, p):
            return pltpu.make_async_copy(
                cos_hbm.at[pl.ds(row0, sB)], bufs_in[p].at[k],
                in_sems[p].at[k])

        def out_cp(row0, k, p):
            return pltpu.make_async_copy(
                bufs_out[p].at[k], out_hbm.at[pl.ds(row0, sB)],
                out_sems[p].at[k])

        @pl.when(g == 0)
        def _prime():
            for k in range(K):
                for p in range(P):
                    in_cp((k * P + p) * sB, k, p).start(priority=p % 2)

        for k in range(K):
            for p in range(P):
                row0 = ((g * K + k) * P + p) * sB
                in_cp(row0, k, p).wait()

                @pl.when(g > 0)
                def _drain_out():
                    out_cp(row0 - rows_per_step, k, p).wait()

                r0 = (k * P + p) * sB
                x = bufs_in[p][k]
                lab = lab_ref[pl.ds(r0, sB), :]
                fv = fix_ref[pl.ds(r0, sB), :]
                cols = lax.broadcasted_iota(jnp.int32, (1, C), 1)
                bufs_out[p][k] = jnp.where(cols == lab, fv, x * _S)
                out_cp(row0, k, p).start(priority=p % 2)

                @pl.when(g + 1 < nsteps)
                def _prefetch():
                    in_cp(row0 + rows_per_step, k, p).start(priority=p % 2)

        @pl.when(g == nsteps - 1)
        def _drain_all():
            for k in range(K):
                for p in range(P):
                    out_cp(((g * K + k) * P + p) * sB, k, p).wait()

    return pl.pallas_call(
        body,
        grid=(nsteps,),
        in_specs=[
            pl.BlockSpec(memory_space=pl.ANY),
            pl.BlockSpec((rows_per_step, 1), lambda i: (i, 0)),
            pl.BlockSpec((rows_per_step, 1), lambda i: (i, 0)),
        ],
        out_specs=pl.BlockSpec(memory_space=pl.ANY),
        out_shape=jax.ShapeDtypeStruct((B, C), jnp.float32),
        scratch_shapes=(
            [pltpu.VMEM((K, sB, C), jnp.float32) for _ in range(2 * P)]
            + [pltpu.SemaphoreType.DMA((K,)) for _ in range(2 * P)]
        ),
    )(cosine, label2d, fix2d)


def kernel(cosine, label):
    B, C = cosine.shape
    margin = jax.random.normal(jax.random.key(42), (B,), jnp.float32) * 0.1 + _M
    cms = jnp.cos(margin) * _S
    sms = jnp.sin(margin) * _S
    fix = _sc_fix_vals(cosine, label, cms, sms)
    return _tc_apply(cosine, label.reshape(B, 1), fix.reshape(B, 1))
